# baseline (speedup 1.0000x reference)
import jax
import jax.numpy as jnp
from jax.experimental import pallas as pl
from jax.experimental.pallas import tpu as pltpu

B, D, H, E, TOPK = 16384, 2048, 256, 64, 3
BT = 512

NEG_INF = float("-inf")


def _router_body(x_ref, w1_ref, b1_ref, w2_ref, b2_ref, w3_ref, b3_ref,
                 idx_ref, wgt_ref):
    x = x_ref[...]
    h = jnp.dot(x, w1_ref[...], preferred_element_type=jnp.float32)
    h = jnp.maximum(h + b1_ref[...], 0.0)
    h = jnp.dot(h.astype(jnp.bfloat16), w2_ref[...],
                preferred_element_type=jnp.float32)
    h = jnp.maximum(h + b2_ref[...], 0.0)
    s = jnp.dot(h.astype(jnp.bfloat16), w3_ref[...],
                preferred_element_type=jnp.float32)
    s = s + b3_ref[...]

    lane = jax.lax.broadcasted_iota(jnp.int32, (BT, E), 1)
    vals = []
    for k in range(TOPK):
        m = jnp.max(s, axis=1, keepdims=True)
        a = jnp.min(jnp.where(s == m, lane, E), axis=1, keepdims=True)
        idx_ref[:, k:k + 1] = a
        vals.append(m)
        s = jnp.where(lane == a, NEG_INF, s)

    e1 = jnp.exp(vals[1] - vals[0])
    e2 = jnp.exp(vals[2] - vals[0])
    denom = 1.0 + e1 + e2
    wgt_ref[:, 0:1] = 1.0 / denom
    wgt_ref[:, 1:2] = e1 / denom
    wgt_ref[:, 2:3] = e2 / denom


@jax.jit
def kernel(prompt_embedding, W1, b1, W2, b2, W3, b3):
    xb = prompt_embedding.astype(jnp.bfloat16)
    grid = (B // BT,)
    idx, wgt = pl.pallas_call(
        _router_body,
        grid=grid,
        in_specs=[
            pl.BlockSpec((BT, D), lambda i: (i, 0)),
            pl.BlockSpec((D, H), lambda i: (0, 0)),
            pl.BlockSpec((1, H), lambda i: (0, 0)),
            pl.BlockSpec((H, H), lambda i: (0, 0)),
            pl.BlockSpec((1, H), lambda i: (0, 0)),
            pl.BlockSpec((H, E), lambda i: (0, 0)),
            pl.BlockSpec((1, E), lambda i: (0, 0)),
        ],
        out_specs=[
            pl.BlockSpec((BT, TOPK), lambda i: (i, 0)),
            pl.BlockSpec((BT, TOPK), lambda i: (i, 0)),
        ],
        out_shape=[
            jax.ShapeDtypeStruct((B, TOPK), jnp.int32),
            jax.ShapeDtypeStruct((B, TOPK), jnp.float32),
        ],
        compiler_params=pltpu.CompilerParams(
            dimension_semantics=("arbitrary",),
        ),
    )(xb, W1.astype(jnp.bfloat16), b1.reshape(1, H),
      W2.astype(jnp.bfloat16), b2.reshape(1, H),
      W3.astype(jnp.bfloat16), b3.reshape(1, E))
    return (idx, wgt, idx[:, 0])


# speedup vs baseline: 1.5739x; 1.5739x over previous
import jax
import jax.numpy as jnp
from jax.experimental import pallas as pl
from jax.experimental.pallas import tpu as pltpu

B, D, H, E, TOPK = 16384, 2048, 256, 64, 3
BT = 512

NEG_INF = float("-inf")


def _router_body(x_ref, w1_ref, b1_ref, w2_ref, b2_ref, w3_ref, b3_ref,
                 idx_ref, wgt_ref):
    x = x_ref[...].astype(jnp.bfloat16)
    h = jnp.dot(x, w1_ref[...], preferred_element_type=jnp.float32)
    h = jnp.maximum(h + b1_ref[...], 0.0)
    h = jnp.dot(h.astype(jnp.bfloat16), w2_ref[...],
                preferred_element_type=jnp.float32)
    h = jnp.maximum(h + b2_ref[...], 0.0)
    s = jnp.dot(h.astype(jnp.bfloat16), w3_ref[...],
                preferred_element_type=jnp.float32)
    s = s + b3_ref[...]

    lane = jax.lax.broadcasted_iota(jnp.int32, (BT, E), 1)
    vals = []
    for k in range(TOPK):
        m = jnp.max(s, axis=1, keepdims=True)
        a = jnp.min(jnp.where(s == m, lane, E), axis=1, keepdims=True)
        idx_ref[:, k:k + 1] = a
        vals.append(m)
        s = jnp.where(lane == a, NEG_INF, s)

    e1 = jnp.exp(vals[1] - vals[0])
    e2 = jnp.exp(vals[2] - vals[0])
    denom = 1.0 + e1 + e2
    wgt_ref[:, 0:1] = 1.0 / denom
    wgt_ref[:, 1:2] = e1 / denom
    wgt_ref[:, 2:3] = e2 / denom


@jax.jit
def kernel(prompt_embedding, W1, b1, W2, b2, W3, b3):
    grid = (B // BT,)
    idx, wgt = pl.pallas_call(
        _router_body,
        grid=grid,
        in_specs=[
            pl.BlockSpec((BT, D), lambda i: (i, 0)),
            pl.BlockSpec((D, H), lambda i: (0, 0)),
            pl.BlockSpec((1, H), lambda i: (0, 0)),
            pl.BlockSpec((H, H), lambda i: (0, 0)),
            pl.BlockSpec((1, H), lambda i: (0, 0)),
            pl.BlockSpec((H, E), lambda i: (0, 0)),
            pl.BlockSpec((1, E), lambda i: (0, 0)),
        ],
        out_specs=[
            pl.BlockSpec((BT, TOPK), lambda i: (i, 0)),
            pl.BlockSpec((BT, TOPK), lambda i: (i, 0)),
        ],
        out_shape=[
            jax.ShapeDtypeStruct((B, TOPK), jnp.int32),
            jax.ShapeDtypeStruct((B, TOPK), jnp.float32),
        ],
        compiler_params=pltpu.CompilerParams(
            dimension_semantics=("parallel",),
        ),
    )(prompt_embedding, W1.astype(jnp.bfloat16), b1.reshape(1, H),
      W2.astype(jnp.bfloat16), b2.reshape(1, H),
      W3.astype(jnp.bfloat16), b3.reshape(1, E))
    return (idx, wgt, idx[:, 0])


# speedup vs baseline: 1.8246x; 1.1592x over previous
import jax
import jax.numpy as jnp
from jax.experimental import pallas as pl
from jax.experimental.pallas import tpu as pltpu

B, D, H, E, TOPK = 16384, 2048, 256, 64, 3
BT = 1024

NEG_INF = float("-inf")


def _router_body(x_ref, w1_ref, b1_ref, w2_ref, b2_ref, w3_ref, b3_ref,
                 idx_ref, wgt_ref):
    x = x_ref[...].astype(jnp.bfloat16)
    h = jnp.dot(x, w1_ref[...], preferred_element_type=jnp.float32)
    h = jnp.maximum(h + b1_ref[...], 0.0)
    h = jnp.dot(h.astype(jnp.bfloat16), w2_ref[...],
                preferred_element_type=jnp.float32)
    h = jnp.maximum(h + b2_ref[...], 0.0)
    s = jnp.dot(h.astype(jnp.bfloat16), w3_ref[...],
                preferred_element_type=jnp.float32)
    s = s + b3_ref[...]

    lane = jax.lax.broadcasted_iota(jnp.int32, (BT, E), 1)
    bits = jax.lax.bitcast_convert_type(s, jnp.int32)
    key = bits ^ (jnp.int32(0x7FFFFFFF) & (bits >> 31))
    key = (key & jnp.int32(-64)) | (jnp.int32(E - 1) - lane)
    vals = []
    for k in range(TOPK):
        m = jnp.max(key, axis=1, keepdims=True)
        idx_ref[:, k:k + 1] = jnp.int32(E - 1) - (m & jnp.int32(E - 1))
        mv = m & jnp.int32(-64)
        vbits = mv ^ (jnp.int32(0x7FFFFFFF) & (mv >> 31))
        vals.append(jax.lax.bitcast_convert_type(vbits, jnp.float32))
        if k + 1 < TOPK:
            key = jnp.where(key == m, jnp.iinfo(jnp.int32).min, key)

    e1 = jnp.exp(vals[1] - vals[0])
    e2 = jnp.exp(vals[2] - vals[0])
    denom = 1.0 + e1 + e2
    wgt_ref[:, 0:1] = 1.0 / denom
    wgt_ref[:, 1:2] = e1 / denom
    wgt_ref[:, 2:3] = e2 / denom


@jax.jit
def kernel(prompt_embedding, W1, b1, W2, b2, W3, b3):
    grid = (B // BT,)
    idx, wgt = pl.pallas_call(
        _router_body,
        grid=grid,
        in_specs=[
            pl.BlockSpec((BT, D), lambda i: (i, 0)),
            pl.BlockSpec((D, H), lambda i: (0, 0)),
            pl.BlockSpec((1, H), lambda i: (0, 0)),
            pl.BlockSpec((H, H), lambda i: (0, 0)),
            pl.BlockSpec((1, H), lambda i: (0, 0)),
            pl.BlockSpec((H, E), lambda i: (0, 0)),
            pl.BlockSpec((1, E), lambda i: (0, 0)),
        ],
        out_specs=[
            pl.BlockSpec((BT, TOPK), lambda i: (i, 0)),
            pl.BlockSpec((BT, TOPK), lambda i: (i, 0)),
        ],
        out_shape=[
            jax.ShapeDtypeStruct((B, TOPK), jnp.int32),
            jax.ShapeDtypeStruct((B, TOPK), jnp.float32),
        ],
        compiler_params=pltpu.CompilerParams(
            dimension_semantics=("parallel",),
        ),
    )(prompt_embedding, W1.astype(jnp.bfloat16), b1.reshape(1, H),
      W2.astype(jnp.bfloat16), b2.reshape(1, H),
      W3.astype(jnp.bfloat16), b3.reshape(1, E))
    return (idx, wgt, idx[:, 0])


# speedup vs baseline: 2.0178x; 1.1059x over previous
import jax
import jax.numpy as jnp
from jax.experimental import pallas as pl
from jax.experimental.pallas import tpu as pltpu

B, D, H, E, TOPK = 16384, 2048, 256, 64, 3
BT = 1024

NEG_INF = float("-inf")


def _router_body(x_ref, w1_ref, b1_ref, w2_ref, b2_ref, w3_ref, b3_ref,
                 idx_ref, wgt_ref, prim_ref, w1b, w2b, w3b):
    LOW6 = jnp.int32(E - 1)
    CLEAR6 = jnp.int32(-E)
    @pl.when(pl.program_id(0) == 0)
    def _cast_weights():
        w1b[...] = w1_ref[...].astype(jnp.bfloat16)
        w2b[...] = w2_ref[...].astype(jnp.bfloat16)
        w3b[...] = w3_ref[...].astype(jnp.bfloat16)

    x = x_ref[...].astype(jnp.bfloat16)
    h = jnp.dot(x, w1b[...], preferred_element_type=jnp.float32)
    h = jnp.maximum(h + b1_ref[...], 0.0)
    h = jnp.dot(h.astype(jnp.bfloat16), w2b[...],
                preferred_element_type=jnp.float32)
    h = jnp.maximum(h + b2_ref[...], 0.0)
    s = jnp.dot(h.astype(jnp.bfloat16), w3b[...],
                preferred_element_type=jnp.float32)
    s = s + b3_ref[...]

    lane = jax.lax.broadcasted_iota(jnp.int32, (BT, E), 1)
    bits = jax.lax.bitcast_convert_type(s, jnp.int32)
    packed = jnp.where(bits < 0, lane, LOW6 - lane)
    key = jax.lax.bitcast_convert_type((bits & CLEAR6) | packed, jnp.float32)
    vals = []
    for k in range(TOPK):
        m = jnp.max(key, axis=1, keepdims=True)
        mb = jax.lax.bitcast_convert_type(m, jnp.int32)
        low = mb & LOW6
        idx = jnp.where(mb < 0, low, LOW6 - low)
        idx_ref[:, k:k + 1] = idx
        if k == 0:
            prim_ref[...] = idx
        vals.append(jax.lax.bitcast_convert_type(mb & CLEAR6, jnp.float32))
        if k + 1 < TOPK:
            key = jnp.where(key == m, NEG_INF, key)

    e1 = jnp.exp(vals[1] - vals[0])
    e2 = jnp.exp(vals[2] - vals[0])
    denom = 1.0 + e1 + e2
    wgt_ref[:, 0:1] = 1.0 / denom
    wgt_ref[:, 1:2] = e1 / denom
    wgt_ref[:, 2:3] = e2 / denom


@jax.jit
def kernel(prompt_embedding, W1, b1, W2, b2, W3, b3):
    grid = (B // BT,)
    idx, wgt, prim = pl.pallas_call(
        _router_body,
        grid=grid,
        in_specs=[
            pl.BlockSpec((BT, D), lambda i: (i, 0)),
            pl.BlockSpec((D, H), lambda i: (0, 0)),
            pl.BlockSpec((1, H), lambda i: (0, 0)),
            pl.BlockSpec((H, H), lambda i: (0, 0)),
            pl.BlockSpec((1, H), lambda i: (0, 0)),
            pl.BlockSpec((H, E), lambda i: (0, 0)),
            pl.BlockSpec((1, E), lambda i: (0, 0)),
        ],
        out_specs=[
            pl.BlockSpec((BT, TOPK), lambda i: (i, 0)),
            pl.BlockSpec((BT, TOPK), lambda i: (i, 0)),
            pl.BlockSpec((BT, 1), lambda i: (i, 0)),
        ],
        out_shape=[
            jax.ShapeDtypeStruct((B, TOPK), jnp.int32),
            jax.ShapeDtypeStruct((B, TOPK), jnp.float32),
            jax.ShapeDtypeStruct((B, 1), jnp.int32),
        ],
        scratch_shapes=[
            pltpu.VMEM((D, H), jnp.bfloat16),
            pltpu.VMEM((H, H), jnp.bfloat16),
            pltpu.VMEM((H, E), jnp.bfloat16),
        ],
        compiler_params=pltpu.CompilerParams(
            dimension_semantics=("arbitrary",),
        ),
    )(prompt_embedding, W1, b1.reshape(1, H), W2, b2.reshape(1, H),
      W3, b3.reshape(1, E))
    return (idx, wgt, prim.reshape(B))
